# baseline (speedup 1.0000x reference)
import numpy as np
import jax
import jax.numpy as jnp
from jax.experimental import pallas as pl
from jax.experimental.pallas import tpu as pltpu

_N = 50
_M = 56
_PL = 128
_ROWS = 16384
_COLS = 100
_BN = 16384


def _build_consts():
    n = _N
    np.random.seed(40)
    xs = np.random.uniform(0, 3, n)
    np.random.seed(122)
    ys = np.random.uniform(0, 3, n)
    np.random.seed(36)
    noise = np.random.normal(0, 1, n)
    xs = np.asarray(xs, np.float32)
    ys = np.asarray(ys, np.float32)
    two = np.float32(2)
    four = np.float32(4)
    term1 = two * np.cos(two * xs) - (xs + ys) * four * np.sin(two * xs)
    term2 = two * np.cos(two * ys) - (xs + ys) * four * np.sin(two * ys)
    f_obs = (term1 + term2 + np.asarray(noise, np.float32)).astype(np.float32)
    u_x = (np.cos(two * xs) * two).astype(np.float32)
    u_y = (np.cos(two * ys) * two).astype(np.float32)

    crow = np.zeros((8, _PL), np.float32)
    crow[0, :n] = xs
    crow[1, :n] = ys

    ccol = np.zeros((5 * _M, _PL), np.float32)
    for k, arr in enumerate((xs, ys, u_x, u_y, f_obs)):
        ccol[k * _M:k * _M + n, :] = arr[:, None]
    return jnp.asarray(crow), jnp.asarray(ccol)


_CROW, _CCOL = None, None


def _consts():
    global _CROW, _CCOL
    if _CROW is None:
        _CROW, _CCOL = _build_consts()
    return _CROW, _CCOL


def _nn_derivative_column(coord_col, coord_row, ku_col, jj, ii):
    ku_row = jnp.sum(jnp.where(ii == jj, ku_col, 0.0), axis=0, keepdims=True)
    diff = coord_col - coord_row
    dist = jnp.abs(diff)
    dist = jnp.where(jj == ii, 1e8, dist)
    dist = jnp.where(jj >= _N, 3e9, dist)
    min_d = jnp.min(dist, axis=1, keepdims=True)
    big_j = jnp.int32(2 ** 30)
    idx = jnp.min(jnp.where(dist == min_d, jj, big_j), axis=1, keepdims=True)
    onehot = (jj == idx).astype(jnp.float32)
    ku_nbr = jnp.sum(onehot * ku_row, axis=1, keepdims=True)
    d_nbr = jnp.sum(onehot * diff, axis=1, keepdims=True)
    return (ku_col - ku_nbr) / (d_nbr + 1e-8)


def _body(crow_ref, ccol_ref, xq_ref, xa_ref, xb_ref, out_ref, u_scr):
    @pl.when(pl.program_id(0) == 0)
    def _():
        jj = jax.lax.broadcasted_iota(jnp.int32, (_M, _PL), 1)
        ii = jax.lax.broadcasted_iota(jnp.int32, (_M, _PL), 0)

        q_col = xq_ref[0:_M, 0:1]
        q_col = jnp.clip(q_col, -10.0, 10.0)
        xs_row = crow_ref[0:1, :]
        ys_row = crow_ref[1:2, :]
        xs_col = ccol_ref[0:_M, :]
        ys_col = ccol_ref[_M:2 * _M, :]
        u_x_col = ccol_ref[2 * _M:3 * _M, 0:1]
        u_y_col = ccol_ref[3 * _M:4 * _M, 0:1]
        f_obs_col = ccol_ref[4 * _M:5 * _M, 0:1]

        ku_x_col = jnp.clip(q_col * u_x_col, -1e6, 1e6)
        ku_y_col = jnp.clip(q_col * u_y_col, -1e6, 1e6)

        d_ku_dx = _nn_derivative_column(xs_col, xs_row, ku_x_col, jj, ii)
        d_ku_dy = _nn_derivative_column(ys_col, ys_row, ku_y_col, jj, ii)
        f_hat = jnp.clip(d_ku_dx + d_ku_dy, -200.0, 200.0)
        diff = f_obs_col - f_hat
        ii_col = jax.lax.broadcasted_iota(jnp.int32, (_M, 1), 0)
        diff = jnp.where(ii_col < _N, diff, 0.0)
        u_scr[0, 0] = 0.5 * jnp.sum(diff * diff)

    xa = xa_ref[...]
    ra = jax.lax.broadcasted_iota(jnp.int32, (48, _BN), 0)
    sa = jnp.sum(jnp.where(ra >= 2, xa * xa, 0.0), axis=0, keepdims=True)
    xb = xb_ref[...]
    rb = jax.lax.broadcasted_iota(jnp.int32, (8, _BN), 0)
    sb = jnp.sum(jnp.where(rb < 4, xb * xb, 0.0), axis=0, keepdims=True)
    val = 0.5 * (sa + sb) + u_scr[0, 0]
    out_ref[...] = jnp.reshape(val, (_BN // 128, 128))


def kernel(x):
    crow, ccol = _consts()
    xt = x.T
    grid = (_ROWS // _BN,)
    out = pl.pallas_call(
        _body,
        grid=grid,
        in_specs=[
            pl.BlockSpec((8, _PL), lambda j: (0, 0)),
            pl.BlockSpec((5 * _M, _PL), lambda j: (0, 0)),
            pl.BlockSpec((_M, 128), lambda j: (0, 0)),
            pl.BlockSpec((48, _BN), lambda j: (1, j)),
            pl.BlockSpec((8, _BN), lambda j: (12, j)),
        ],
        out_specs=pl.BlockSpec((_BN // 128, 128), lambda j: (j, 0)),
        out_shape=jax.ShapeDtypeStruct((_ROWS // 128, 128), jnp.float32),
        scratch_shapes=[pltpu.SMEM((1, 1), jnp.float32)],
    )(crow, ccol, xt, xt, xt)
    return jnp.reshape(out, (_ROWS, 1))


# speedup vs baseline: 1.0350x; 1.0350x over previous
import numpy as np
import jax
import jax.numpy as jnp
from jax.experimental import pallas as pl
from jax.experimental.pallas import tpu as pltpu

_N = 50
_M = 56
_PL = 128
_ROWS = 16384
_COLS = 100
_BN = 8192


def _build_consts():
    n = _N
    np.random.seed(40)
    xs = np.random.uniform(0, 3, n)
    np.random.seed(122)
    ys = np.random.uniform(0, 3, n)
    np.random.seed(36)
    noise = np.random.normal(0, 1, n)
    xs = np.asarray(xs, np.float32)
    ys = np.asarray(ys, np.float32)
    two = np.float32(2)
    four = np.float32(4)
    term1 = two * np.cos(two * xs) - (xs + ys) * four * np.sin(two * xs)
    term2 = two * np.cos(two * ys) - (xs + ys) * four * np.sin(two * ys)
    f_obs = (term1 + term2 + np.asarray(noise, np.float32)).astype(np.float32)
    u_x = (np.cos(two * xs) * two).astype(np.float32)
    u_y = (np.cos(two * ys) * two).astype(np.float32)

    crow = np.zeros((8, _PL), np.float32)
    crow[0, :n] = xs
    crow[1, :n] = ys

    ccol = np.zeros((5 * _M, _PL), np.float32)
    for k, arr in enumerate((xs, ys, u_x, u_y, f_obs)):
        ccol[k * _M:k * _M + n, :] = arr[:, None]
    return jnp.asarray(crow), jnp.asarray(ccol)


_CROW, _CCOL = None, None


def _consts():
    global _CROW, _CCOL
    if _CROW is None:
        _CROW, _CCOL = _build_consts()
    return _CROW, _CCOL


def _nn_derivative_column(coord_col, coord_row, ku_col, jj, ii):
    ku_row = jnp.sum(jnp.where(ii == jj, ku_col, 0.0), axis=0, keepdims=True)
    diff = coord_col - coord_row
    dist = jnp.abs(diff)
    dist = jnp.where(jj == ii, 1e8, dist)
    dist = jnp.where(jj >= _N, 3e9, dist)
    min_d = jnp.min(dist, axis=1, keepdims=True)
    big_j = jnp.int32(2 ** 30)
    idx = jnp.min(jnp.where(dist == min_d, jj, big_j), axis=1, keepdims=True)
    onehot = (jj == idx).astype(jnp.float32)
    ku_nbr = jnp.sum(onehot * ku_row, axis=1, keepdims=True)
    d_nbr = jnp.sum(onehot * diff, axis=1, keepdims=True)
    return (ku_col - ku_nbr) / (d_nbr + 1e-8)


def _body(crow_ref, ccol_ref, xq_ref, xa_ref, xb_ref, out_ref, u_scr):
    @pl.when(pl.program_id(0) == 0)
    def _():
        jj = jax.lax.broadcasted_iota(jnp.int32, (_M, _PL), 1)
        ii = jax.lax.broadcasted_iota(jnp.int32, (_M, _PL), 0)

        q_col = xq_ref[0:_M, 0:1]
        q_col = jnp.clip(q_col, -10.0, 10.0)
        xs_row = crow_ref[0:1, :]
        ys_row = crow_ref[1:2, :]
        xs_col = ccol_ref[0:_M, :]
        ys_col = ccol_ref[_M:2 * _M, :]
        u_x_col = ccol_ref[2 * _M:3 * _M, 0:1]
        u_y_col = ccol_ref[3 * _M:4 * _M, 0:1]
        f_obs_col = ccol_ref[4 * _M:5 * _M, 0:1]

        ku_x_col = jnp.clip(q_col * u_x_col, -1e6, 1e6)
        ku_y_col = jnp.clip(q_col * u_y_col, -1e6, 1e6)

        d_ku_dx = _nn_derivative_column(xs_col, xs_row, ku_x_col, jj, ii)
        d_ku_dy = _nn_derivative_column(ys_col, ys_row, ku_y_col, jj, ii)
        f_hat = jnp.clip(d_ku_dx + d_ku_dy, -200.0, 200.0)
        diff = f_obs_col - f_hat
        ii_col = jax.lax.broadcasted_iota(jnp.int32, (_M, 1), 0)
        diff = jnp.where(ii_col < _N, diff, 0.0)
        u_scr[0, 0] = 0.5 * jnp.sum(diff * diff)

    xa = xa_ref[...]
    ra = jax.lax.broadcasted_iota(jnp.int32, (48, _BN), 0)
    sa = jnp.sum(jnp.where(ra >= 2, xa * xa, 0.0), axis=0, keepdims=True)
    xb = xb_ref[...]
    rb = jax.lax.broadcasted_iota(jnp.int32, (8, _BN), 0)
    sb = jnp.sum(jnp.where(rb < 4, xb * xb, 0.0), axis=0, keepdims=True)
    val = 0.5 * (sa + sb) + u_scr[0, 0]
    out_ref[...] = jnp.reshape(val, (_BN // 128, 128))


def kernel(x):
    crow, ccol = _consts()
    xt = x.T
    grid = (_ROWS // _BN,)
    out = pl.pallas_call(
        _body,
        grid=grid,
        in_specs=[
            pl.BlockSpec((8, _PL), lambda j: (0, 0)),
            pl.BlockSpec((5 * _M, _PL), lambda j: (0, 0)),
            pl.BlockSpec((_M, 128), lambda j: (0, 0)),
            pl.BlockSpec((48, _BN), lambda j: (1, j)),
            pl.BlockSpec((8, _BN), lambda j: (12, j)),
        ],
        out_specs=pl.BlockSpec((_BN // 128, 128), lambda j: (j, 0)),
        out_shape=jax.ShapeDtypeStruct((_ROWS // 128, 128), jnp.float32),
        scratch_shapes=[pltpu.SMEM((1, 1), jnp.float32)],
    )(crow, ccol, xt, xt, xt)
    return jnp.reshape(out, (_ROWS, 1))


# speedup vs baseline: 1.0613x; 1.0255x over previous
import numpy as np
import jax
import jax.numpy as jnp
from jax.experimental import pallas as pl
from jax.experimental.pallas import tpu as pltpu

_N = 50
_M = 56
_PL = 128
_ROWS = 16384
_COLS = 100
_BN = 8192


def _build_consts():
    n = _N
    np.random.seed(40)
    xs = np.random.uniform(0, 3, n)
    np.random.seed(122)
    ys = np.random.uniform(0, 3, n)
    np.random.seed(36)
    noise = np.random.normal(0, 1, n)
    xs = np.asarray(xs, np.float32)
    ys = np.asarray(ys, np.float32)
    two = np.float32(2)
    four = np.float32(4)
    term1 = two * np.cos(two * xs) - (xs + ys) * four * np.sin(two * xs)
    term2 = two * np.cos(two * ys) - (xs + ys) * four * np.sin(two * ys)
    f_obs = (term1 + term2 + np.asarray(noise, np.float32)).astype(np.float32)
    u_x = (np.cos(two * xs) * two).astype(np.float32)
    u_y = (np.cos(two * ys) * two).astype(np.float32)

    crow = np.zeros((8, _PL), np.float32)
    crow[0, :n] = xs
    crow[1, :n] = ys

    ccol = np.zeros((5 * _M, _PL), np.float32)
    for k, arr in enumerate((xs, ys, u_x, u_y, f_obs)):
        ccol[k * _M:k * _M + n, :] = arr[:, None]
    return jnp.asarray(crow), jnp.asarray(ccol)


_CROW, _CCOL = None, None


def _consts():
    global _CROW, _CCOL
    if _CROW is None:
        _CROW, _CCOL = _build_consts()
    return _CROW, _CCOL


def _nn_derivative_column(coord_col, coord_row, ku_col, jj, ii):
    ku_row = jnp.sum(jnp.where(ii == jj, ku_col, 0.0), axis=0, keepdims=True)
    diff = coord_col - coord_row
    dist = jnp.abs(diff)
    dist = jnp.where(jj == ii, 1e8, dist)
    dist = jnp.where(jj >= _N, 3e9, dist)
    min_d = jnp.min(dist, axis=1, keepdims=True)
    big_j = jnp.int32(2 ** 30)
    idx = jnp.min(jnp.where(dist == min_d, jj, big_j), axis=1, keepdims=True)
    onehot = (jj == idx).astype(jnp.float32)
    ku_nbr = jnp.sum(onehot * ku_row, axis=1, keepdims=True)
    d_nbr = jnp.sum(onehot * diff, axis=1, keepdims=True)
    return (ku_col - ku_nbr) / (d_nbr + 1e-8)


def _body(crow_ref, ccol_ref, xq_ref, xa_ref, xa2_ref, xb_ref, out_ref, u_scr):
    @pl.when(pl.program_id(0) == 0)
    def _():
        jj = jax.lax.broadcasted_iota(jnp.int32, (_M, _PL), 1)
        ii = jax.lax.broadcasted_iota(jnp.int32, (_M, _PL), 0)

        q_col = xq_ref[0:_M, 0:1]
        q_col = jnp.clip(q_col, -10.0, 10.0)
        xs_row = crow_ref[0:1, :]
        ys_row = crow_ref[1:2, :]
        xs_col = ccol_ref[0:_M, :]
        ys_col = ccol_ref[_M:2 * _M, :]
        u_x_col = ccol_ref[2 * _M:3 * _M, 0:1]
        u_y_col = ccol_ref[3 * _M:4 * _M, 0:1]
        f_obs_col = ccol_ref[4 * _M:5 * _M, 0:1]

        ku_x_col = jnp.clip(q_col * u_x_col, -1e6, 1e6)
        ku_y_col = jnp.clip(q_col * u_y_col, -1e6, 1e6)

        d_ku_dx = _nn_derivative_column(xs_col, xs_row, ku_x_col, jj, ii)
        d_ku_dy = _nn_derivative_column(ys_col, ys_row, ku_y_col, jj, ii)
        f_hat = jnp.clip(d_ku_dx + d_ku_dy, -200.0, 200.0)
        diff = f_obs_col - f_hat
        ii_col = jax.lax.broadcasted_iota(jnp.int32, (_M, 1), 0)
        diff = jnp.where(ii_col < _N, diff, 0.0)
        u_scr[0, 0] = 0.5 * jnp.sum(diff * diff)

    xa = xa_ref[...]
    ra = jax.lax.broadcasted_iota(jnp.int32, (24, _BN), 0)
    sa = jnp.sum(jnp.where(ra >= 2, xa * xa, 0.0), axis=0, keepdims=True)
    xa2 = xa2_ref[...]
    sa = sa + jnp.sum(xa2 * xa2, axis=0, keepdims=True)
    xb = xb_ref[...]
    rb = jax.lax.broadcasted_iota(jnp.int32, (8, _BN), 0)
    sb = jnp.sum(jnp.where(rb < 4, xb * xb, 0.0), axis=0, keepdims=True)
    val = 0.5 * (sa + sb) + u_scr[0, 0]
    out_ref[...] = jnp.reshape(val, (_BN // 128, 128))


def kernel(x):
    crow, ccol = _consts()
    xt = x.T
    grid = (_ROWS // _BN,)
    out = pl.pallas_call(
        _body,
        grid=grid,
        in_specs=[
            pl.BlockSpec((8, _PL), lambda j: (0, 0)),
            pl.BlockSpec((5 * _M, _PL), lambda j: (0, 0)),
            pl.BlockSpec((_M, 128), lambda j: (0, 0)),
            pl.BlockSpec((24, _BN), lambda j: (2, j)),
            pl.BlockSpec((24, _BN), lambda j: (3, j)),
            pl.BlockSpec((8, _BN), lambda j: (12, j)),
        ],
        out_specs=pl.BlockSpec((_BN // 128, 128), lambda j: (j, 0)),
        out_shape=jax.ShapeDtypeStruct((_ROWS // 128, 128), jnp.float32),
        scratch_shapes=[pltpu.SMEM((1, 1), jnp.float32)],
    )(crow, ccol, xt, xt, xt, xt)
    return jnp.reshape(out, (_ROWS, 1))
